# baseline (speedup 1.0000x reference)
import jax
import jax.numpy as jnp
from jax.experimental import pallas as pl
from jax.experimental.pallas import tpu as pltpu


def _x1t_kernel(x_ref, w1_ref, b1_ref, out_ref):
    blk = x_ref.shape[0]
    x1 = jnp.dot(x_ref[...], w1_ref[...],
                 preferred_element_type=jnp.float32) + b1_ref[...]
    x1aug = jnp.concatenate(
        [x1, jnp.ones((blk, 8), jnp.float32)], axis=1)
    out_ref[0, :, :] = jnp.transpose(x1aug).astype(jnp.bfloat16)


def _fused_kernel(x1t_ref, hg_ref, w2_ref, b2_ref, out_ref, acc_ref, y_ref):
    s = pl.program_id(0)
    n = pl.program_id(1)
    nsteps = pl.num_programs(1)
    C = w2_ref.shape[0]

    @pl.when(s == 0)
    def _v2e():
        part = jax.lax.dot_general(
            x1t_ref[0, :, :], hg_ref[...].astype(jnp.bfloat16),
            (((1,), (0,)), ((), ())),
            preferred_element_type=jnp.float32)

        @pl.when(n == 0)
        def _init():
            acc_ref[...] = part

        @pl.when(n > 0)
        def _acc():
            acc_ref[...] += part

        @pl.when(n == nsteps - 1)
        def _finish():
            de = acc_ref[C:C + 1, :]
            y = jnp.maximum(acc_ref[:C, :] * (1.0 / de), 0.0)
            y2 = jax.lax.dot_general(
                w2_ref[...].astype(jnp.bfloat16), y.astype(jnp.bfloat16),
                (((0,), (0,)), ((), ())),
                preferred_element_type=jnp.float32) + b2_ref[...]
            y_ref[...] = jnp.transpose(y2).astype(jnp.bfloat16)

    @pl.when(s == 1)
    def _e2v():
        h = hg_ref[...]
        xv = jnp.dot(h.astype(jnp.bfloat16), y_ref[...],
                     preferred_element_type=jnp.float32)
        dv = jnp.sum(h, axis=1, keepdims=True)
        scale = jnp.where(dv > 0.0, 1.0 / dv, 0.0)
        out_ref[...] = jnp.maximum(xv * scale, 0.0)


@jax.jit
def kernel(X, hg, W_v2e, b_v2e, W_e2v, b_e2v):
    N, C = X.shape
    M = hg.shape[1]
    blk = 1000
    assert N % blk == 0

    b1 = b_v2e.reshape(1, C)
    b2 = b_e2v.reshape(C, 1)

    x1t = pl.pallas_call(
        _x1t_kernel,
        grid=(N // blk,),
        in_specs=[
            pl.BlockSpec((blk, C), lambda n: (n, 0)),
            pl.BlockSpec((C, C), lambda n: (0, 0)),
            pl.BlockSpec((1, C), lambda n: (0, 0)),
        ],
        out_specs=pl.BlockSpec((1, C + 8, blk), lambda n: (n, 0, 0)),
        out_shape=jax.ShapeDtypeStruct((N // blk, C + 8, blk), jnp.bfloat16),
        compiler_params=pltpu.CompilerParams(
            dimension_semantics=("arbitrary",)),
    )(X, W_v2e, b1)

    xv = pl.pallas_call(
        _fused_kernel,
        grid=(2, N // blk),
        in_specs=[
            pl.BlockSpec((1, C + 8, blk), lambda s, n: (n, 0, 0)),
            pl.BlockSpec((blk, M), lambda s, n: (n, 0)),
            pl.BlockSpec((C, C), lambda s, n: (0, 0)),
            pl.BlockSpec((C, 1), lambda s, n: (0, 0)),
        ],
        out_specs=pl.BlockSpec((blk, C), lambda s, n: (n, 0)),
        out_shape=jax.ShapeDtypeStruct((N, C), jnp.float32),
        scratch_shapes=[
            pltpu.VMEM((C + 8, M), jnp.float32),
            pltpu.VMEM((M, C), jnp.bfloat16),
        ],
        compiler_params=pltpu.CompilerParams(
            dimension_semantics=("arbitrary", "arbitrary")),
    )(x1t, hg, W_e2v, b2)

    return xv


# speedup vs baseline: 1.0151x; 1.0151x over previous
import jax
import jax.numpy as jnp
from jax.experimental import pallas as pl
from jax.experimental.pallas import tpu as pltpu


def _fused_kernel(x_ref, hg_ref, w1_ref, b1_ref, w2_ref, b2_ref, out_ref,
                  acc_ref, de_ref, y_ref):
    s = pl.program_id(0)
    n = pl.program_id(1)
    nsteps = pl.num_programs(1)

    @pl.when(s == 0)
    def _v2e():
        h = hg_ref[...]
        x1 = jnp.dot(x_ref[...], w1_ref[...],
                     preferred_element_type=jnp.float32) + b1_ref[...]
        part = jax.lax.dot_general(
            h.astype(jnp.bfloat16), x1.astype(jnp.bfloat16),
            (((0,), (0,)), ((), ())),
            preferred_element_type=jnp.float32)
        de_part = jnp.sum(h, axis=0, keepdims=True)

        @pl.when(n == 0)
        def _init():
            acc_ref[...] = part
            de_ref[...] = de_part

        @pl.when(n > 0)
        def _acc():
            acc_ref[...] += part
            de_ref[...] += de_part

        @pl.when(n == nsteps - 1)
        def _finish():
            scale = jnp.transpose(1.0 / de_ref[...])
            y = jnp.maximum(acc_ref[...] * scale, 0.0)
            y2 = jnp.dot(y.astype(jnp.bfloat16),
                         w2_ref[...].astype(jnp.bfloat16),
                         preferred_element_type=jnp.float32) + b2_ref[...]
            y_ref[...] = y2.astype(jnp.bfloat16)

    @pl.when(s == 1)
    def _e2v():
        h = hg_ref[...]
        xv = jnp.dot(h.astype(jnp.bfloat16), y_ref[...],
                     preferred_element_type=jnp.float32)
        dv = jnp.sum(h, axis=1, keepdims=True)
        scale = jnp.where(dv > 0.0, 1.0 / dv, 0.0)
        out_ref[...] = jnp.maximum(xv * scale, 0.0)


@jax.jit
def kernel(X, hg, W_v2e, b_v2e, W_e2v, b_e2v):
    N, C = X.shape
    M = hg.shape[1]
    blk = 1000
    assert N % blk == 0

    b1 = b_v2e.reshape(1, C)
    b2 = b_e2v.reshape(1, C)

    xv = pl.pallas_call(
        _fused_kernel,
        grid=(2, N // blk),
        in_specs=[
            pl.BlockSpec((blk, C), lambda s, n: (n, 0)),
            pl.BlockSpec((blk, M), lambda s, n: (n, 0)),
            pl.BlockSpec((C, C), lambda s, n: (0, 0)),
            pl.BlockSpec((1, C), lambda s, n: (0, 0)),
            pl.BlockSpec((C, C), lambda s, n: (0, 0)),
            pl.BlockSpec((1, C), lambda s, n: (0, 0)),
        ],
        out_specs=pl.BlockSpec((blk, C), lambda s, n: (n, 0)),
        out_shape=jax.ShapeDtypeStruct((N, C), jnp.float32),
        scratch_shapes=[
            pltpu.VMEM((M, C), jnp.float32),
            pltpu.VMEM((1, M), jnp.float32),
            pltpu.VMEM((M, C), jnp.bfloat16),
        ],
        compiler_params=pltpu.CompilerParams(
            dimension_semantics=("arbitrary", "arbitrary"),
            fuse_transposed_lhs_in_matmul=True),
    )(X, hg, W_v2e, b1, W_e2v, b2)

    return xv
